# baseline (speedup 1.0000x reference)
import functools

import jax
import jax.numpy as jnp
from jax import lax
from jax.experimental import pallas as pl
from jax.experimental.pallas import tpu as pltpu
from jax.experimental.pallas import tpu_sc as plsc

N = 10000
E = 320000
D = 128
H = 64
L = 4
B = 128
EMB = 256

NC = 2
NS = 16
NW = NC * NS

ZR = 16
ROWS_PER_TILE = 640

BLK = 400
NBLK = N // BLK

_LOG2 = 0.6931471805599453


@functools.cache
def _make_segsum(width):
    if width == D:
        CH, SUP, NB, GA = 125, 8, 2, 1
    else:
        CH, SUP, NB, GA = 125, 8, 6, 4
    T_PER_W = E // (NW * CH)
    NSUP = T_PER_W // SUP
    mesh = plsc.VectorSubcoreMesh(
        core_axis_name="c", subcore_axis_name="s", num_cores=NC, num_subcores=NS)

    @functools.partial(
        pl.kernel,
        out_type=jax.ShapeDtypeStruct((2 * N, width), jnp.float32),
        mesh=mesh,
        compiler_params=pltpu.CompilerParams(use_tc_tiling_on_sc=False),
        scratch_types=[
            pltpu.VMEM((2, SUP, CH), jnp.int32),
            pltpu.VMEM((2, SUP, CH), jnp.int32),
            pltpu.VMEM((NB, CH, width), jnp.float32),
            pltpu.VMEM((ZR, width), jnp.float32),
            pltpu.VMEM_SHARED((N, width), jnp.float32),
            pltpu.SemaphoreType.DMA((2,)),
            pltpu.SemaphoreType.DMA((2,)),
            pltpu.SemaphoreType.DMA((NB,)),
            pltpu.SemaphoreType.DMA((NB,)),
        ],
    )
    def segsum(h_hbm, ei_hbm, out_hbm,
               src_v, dst_v, rows_v, zbuf, acc, isem_s, isem_d, gsem, ssem):
        cid = lax.axis_index("c")
        sid = lax.axis_index("s")
        wid = sid * NC + cid
        crow0 = wid * T_PER_W

        def start_idx(s, par):
            r = crow0 + s * SUP
            pltpu.async_copy(ei_hbm.at[0, pl.ds(r, SUP)], src_v.at[par],
                             isem_s.at[par])
            pltpu.async_copy(ei_hbm.at[1, pl.ds(r, SUP)], dst_v.at[par],
                             isem_d.at[par])

        def wait_idx(s, par):
            r = crow0 + s * SUP
            pltpu.make_async_copy(ei_hbm.at[0, pl.ds(r, SUP)], src_v.at[par],
                                  isem_s.at[par]).wait()
            pltpu.make_async_copy(ei_hbm.at[1, pl.ds(r, SUP)], dst_v.at[par],
                                  isem_d.at[par]).wait()

        def start_gather(tg):
            sg = tg // SUP
            parg = sg % 2
            b = tg % NB
            pltpu.async_copy(h_hbm.at[src_v.at[parg, tg % SUP]],
                             rows_v.at[b], gsem.at[b])

        def wait_gather(t):
            b = t % NB
            pltpu.make_async_copy(h_hbm.at[src_v.at[(t // SUP) % 2, t % SUP]],
                                  rows_v.at[b], gsem.at[b]).wait()

        def start_scatter(t):
            b = t % NB
            pltpu.async_copy(rows_v.at[b],
                             acc.at[dst_v.at[(t // SUP) % 2, t % SUP]],
                             ssem.at[b], add=True)

        def wait_scatter(t):
            b = t % NB
            pltpu.make_async_copy(rows_v.at[b],
                                  acc.at[dst_v.at[(t // SUP) % 2, t % SUP]],
                                  ssem.at[b]).wait()

        nvec = ZR * (width // 16)
        z16 = jnp.zeros((16,), jnp.float32)

        def zb_body(t, _):
            r = t // (width // 16)
            c = t % (width // 16)
            zbuf[r, pl.ds(c * 16, 16)] = z16
            return 0
        lax.fori_loop(0, nvec, zb_body, 0)

        row0 = sid * ROWS_PER_TILE
        my_rows = jnp.where(sid == NS - 1, N - (NS - 1) * ROWS_PER_TILE,
                            ROWS_PER_TILE)
        ncopies = my_rows // ZR

        def zc_body(t, _):
            pltpu.sync_copy(zbuf, acc.at[pl.ds(row0 + t * ZR, ZR)])
            return 0
        lax.fori_loop(0, ncopies, zc_body, 0)

        plsc.subcore_barrier()

        start_idx(0, 0)
        wait_idx(0, 0)
        start_idx(1, 1)
        for tp in range(GA):
            start_gather(tp)

        def body(t, _):
            tg = t + GA

            @pl.when(tg < T_PER_W)
            def _():
                @pl.when(tg % SUP == 0)
                def _():
                    sg = tg // SUP
                    wait_idx(sg, sg % 2)

                @pl.when(tg >= NB)
                def _():
                    wait_scatter(tg - NB)
                start_gather(tg)

            @pl.when(jnp.logical_and(t % SUP == NB - GA, t > SUP))
            def _():
                s_next = t // SUP + 1

                @pl.when(s_next < NSUP)
                def _():
                    start_idx(s_next, s_next % 2)

            wait_gather(t)
            start_scatter(t)
            return 0
        lax.fori_loop(0, T_PER_W, body, 0)

        for k in range(NB):
            wait_scatter(T_PER_W - NB + k)

        plsc.subcore_barrier()

        def oc_body(t, _):
            pltpu.sync_copy(acc.at[pl.ds(row0 + t * ZR, ZR)],
                            out_hbm.at[pl.ds(cid * N + row0 + t * ZR, ZR)])
            return 0
        lax.fori_loop(0, ncopies, oc_body, 0)

    return segsum


def _mlp_body(h_ref, a0_ref, a1_ref, w1_ref, b1_ref, w2_ref, b2_ref,
              batch_ref, o_ref, y_ref):
    z = h_ref[...] + a0_ref[...] + a1_ref[...]
    t = jnp.dot(z, w1_ref[...], preferred_element_type=jnp.float32)
    t = jnp.maximum(t + b1_ref[...], 0.0)
    t = jnp.dot(t, w2_ref[...], preferred_element_type=jnp.float32)
    hn = jnp.maximum(t + b2_ref[...], 0.0)
    o_ref[...] = hn
    bt = batch_ref[0]
    ids = lax.broadcasted_iota(jnp.int32, (1, B), 1)
    mask = (bt.reshape(BLK, 1) == ids).astype(jnp.float32)
    contrib = lax.dot_general(mask, hn, (((0,), (0,)), ((), ())),
                              preferred_element_type=jnp.float32)

    @pl.when(pl.program_id(0) == 0)
    def _():
        y_ref[...] = jnp.zeros_like(y_ref)
    y_ref[...] += contrib


def _gin_mlp(h, agg2, batch3, w1, b1, w2, b2):
    din = h.shape[1]
    full = lambda shape: pl.BlockSpec(shape, lambda i: (0, 0))
    return pl.pallas_call(
        _mlp_body,
        grid=(NBLK,),
        in_specs=[
            pl.BlockSpec((BLK, din), lambda i: (i, 0)),
            pl.BlockSpec((BLK, din), lambda i: (i, 0)),
            pl.BlockSpec((BLK, din), lambda i: (i + NBLK, 0)),
            full((din, H)), full((1, H)), full((H, H)), full((1, H)),
            pl.BlockSpec((1, 1, BLK), lambda i: (i, 0, 0)),
        ],
        out_specs=[pl.BlockSpec((BLK, H), lambda i: (i, 0)),
                   pl.BlockSpec((B, H), lambda i: (0, 0))],
        out_shape=[jax.ShapeDtypeStruct((N, H), jnp.float32),
                   jax.ShapeDtypeStruct((B, H), jnp.float32)],
    )(h, agg2, agg2, w1, b1.reshape(1, H), w2, b2.reshape(1, H), batch3)


def _ff(z, w0, b0, w1, b1, w2, b2, ws, bs):
    t = jnp.maximum(jnp.dot(z, w0, preferred_element_type=jnp.float32) + b0, 0.0)
    t = jnp.maximum(jnp.dot(t, w1, preferred_element_type=jnp.float32) + b1, 0.0)
    t = jnp.maximum(jnp.dot(t, w2, preferred_element_type=jnp.float32) + b2, 0.0)
    return t + jnp.dot(z, ws, preferred_element_type=jnp.float32) + bs


def _genc_body(y1, y2, y3, y4, w0, b0, w1, b1, w2, b2, ws, bs, o_ref):
    y = jnp.concatenate([y1[...], y2[...], y3[...], y4[...]], axis=1)
    o_ref[...] = _ff(y, w0[...], b0[...], w1[...], b1[...],
                     w2[...], b2[...], ws[...], bs[...])


def _genc(ys, p):
    args = (*ys, p['ff_g_W0'], p['ff_g_b0'].reshape(1, EMB),
            p['ff_g_W1'], p['ff_g_b1'].reshape(1, EMB),
            p['ff_g_W2'], p['ff_g_b2'].reshape(1, EMB),
            p['ff_g_Ws'], p['ff_g_bs'].reshape(1, EMB))
    return pl.pallas_call(
        _genc_body,
        out_shape=jax.ShapeDtypeStruct((B, EMB), jnp.float32),
    )(*args)


def _loss_body(h1, h2, h3, h4, batch_ref, g_ref,
               w0, b0, w1, b1, w2, b2, ws, bs, o_ref):
    l = jnp.concatenate([h1[...], h2[...], h3[...], h4[...]], axis=1)
    le = _ff(l, w0[...], b0[...], w1[...], b1[...], w2[...], b2[...],
             ws[...], bs[...])
    res = lax.dot_general(le, g_ref[...], (((1,), (1,)), ((), ())),
                          preferred_element_type=jnp.float32)
    bt = batch_ref[0]
    ids = lax.broadcasted_iota(jnp.int32, (1, B), 1)
    m = (bt.reshape(BLK, 1) == ids).astype(jnp.float32)
    sp = jnp.maximum(-res, 0.0) + jnp.log1p(jnp.exp(-jnp.abs(res)))
    pos_sum = jnp.sum((_LOG2 - sp) * m)
    neg_sum = jnp.sum((sp + res - _LOG2) * (1.0 - m))
    contrib = neg_sum / (N * (B - 1)) - pos_sum / N

    @pl.when(pl.program_id(0) == 0)
    def _():
        o_ref[...] = jnp.zeros_like(o_ref)
    o_ref[...] += contrib.reshape(1, 1)


def _loss(h1, h2, h3, h4, batch3, g_enc, p):
    hspec = pl.BlockSpec((BLK, H), lambda i: (i, 0))
    full = lambda shape: pl.BlockSpec(shape, lambda i: (0, 0))
    args = (h1, h2, h3, h4, batch3, g_enc,
            p['ff_l_W0'], p['ff_l_b0'].reshape(1, EMB),
            p['ff_l_W1'], p['ff_l_b1'].reshape(1, EMB),
            p['ff_l_W2'], p['ff_l_b2'].reshape(1, EMB),
            p['ff_l_Ws'], p['ff_l_bs'].reshape(1, EMB))
    return pl.pallas_call(
        _loss_body,
        grid=(NBLK,),
        in_specs=[hspec, hspec, hspec, hspec,
                  pl.BlockSpec((1, 1, BLK), lambda i: (i, 0, 0)),
                  full((B, EMB)),
                  full((EMB, EMB)), full((1, EMB)),
                  full((EMB, EMB)), full((1, EMB)),
                  full((EMB, EMB)), full((1, EMB)),
                  full((EMB, EMB)), full((1, EMB))],
        out_specs=pl.BlockSpec((1, 1), lambda i: (0, 0)),
        out_shape=jax.ShapeDtypeStruct((1, 1), jnp.float32),
    )(*args)


def kernel(x, edge_index, batch, edge_attr, params):
    ei_d = edge_index.reshape(2, E // 125, 125)
    ei_h = ei_d
    batch3 = batch.reshape(NBLK, 1, BLK)

    h = x
    hs = []
    ys = []
    for i in range(L):
        segsum = _make_segsum(D if i == 0 else H)
        agg2 = segsum(h, ei_d if i == 0 else ei_h)
        h, y_i = _gin_mlp(h, agg2, batch3,
                          params['gin_W1_%d' % i], params['gin_b1_%d' % i],
                          params['gin_W2_%d' % i], params['gin_b2_%d' % i])
        hs.append(h)
        ys.append(y_i)

    h1, h2, h3, h4 = hs
    g_enc = _genc(ys, params)
    out = _loss(h1, h2, h3, h4, batch3, g_enc, params)
    return out.reshape(())


# speedup vs baseline: 1.0071x; 1.0071x over previous
import functools

import jax
import jax.numpy as jnp
from jax import lax
from jax.experimental import pallas as pl
from jax.experimental.pallas import tpu as pltpu
from jax.experimental.pallas import tpu_sc as plsc

N = 10000
E = 320000
D = 128
H = 64
L = 4
B = 128
EMB = 256

NC = 2
NS = 16
NW = NC * NS

ZR = 16
ROWS_PER_TILE = 640

BLK = 400
NBLK = N // BLK

_LOG2 = 0.6931471805599453


@functools.cache
def _make_segsum(width):
    if width == D:
        CH, SUP, NB, GA = 100, 10, 3, 2
    else:
        CH, SUP, NB, GA = 125, 8, 6, 4
    T_PER_W = E // (NW * CH)
    NSUP = T_PER_W // SUP
    mesh = plsc.VectorSubcoreMesh(
        core_axis_name="c", subcore_axis_name="s", num_cores=NC, num_subcores=NS)

    @functools.partial(
        pl.kernel,
        out_type=jax.ShapeDtypeStruct((2 * N, width), jnp.float32),
        mesh=mesh,
        compiler_params=pltpu.CompilerParams(use_tc_tiling_on_sc=False),
        scratch_types=[
            pltpu.VMEM((2, SUP, CH), jnp.int32),
            pltpu.VMEM((2, SUP, CH), jnp.int32),
            pltpu.VMEM((NB, CH, width), jnp.float32),
            pltpu.VMEM((ZR, width), jnp.float32),
            pltpu.VMEM_SHARED((N, width), jnp.float32),
            pltpu.SemaphoreType.DMA((2,)),
            pltpu.SemaphoreType.DMA((2,)),
            pltpu.SemaphoreType.DMA((NB,)),
            pltpu.SemaphoreType.DMA((NB,)),
        ],
    )
    def segsum(h_hbm, ei_hbm, out_hbm,
               src_v, dst_v, rows_v, zbuf, acc, isem_s, isem_d, gsem, ssem):
        cid = lax.axis_index("c")
        sid = lax.axis_index("s")
        wid = sid * NC + cid
        crow0 = wid * T_PER_W

        def start_idx(s, par):
            r = crow0 + s * SUP
            pltpu.async_copy(ei_hbm.at[0, pl.ds(r, SUP)], src_v.at[par],
                             isem_s.at[par])
            pltpu.async_copy(ei_hbm.at[1, pl.ds(r, SUP)], dst_v.at[par],
                             isem_d.at[par])

        def wait_idx(s, par):
            r = crow0 + s * SUP
            pltpu.make_async_copy(ei_hbm.at[0, pl.ds(r, SUP)], src_v.at[par],
                                  isem_s.at[par]).wait()
            pltpu.make_async_copy(ei_hbm.at[1, pl.ds(r, SUP)], dst_v.at[par],
                                  isem_d.at[par]).wait()

        def start_gather(tg):
            sg = tg // SUP
            parg = sg % 2
            b = tg % NB
            pltpu.async_copy(h_hbm.at[src_v.at[parg, tg % SUP]],
                             rows_v.at[b], gsem.at[b])

        def wait_gather(t):
            b = t % NB
            pltpu.make_async_copy(h_hbm.at[src_v.at[(t // SUP) % 2, t % SUP]],
                                  rows_v.at[b], gsem.at[b]).wait()

        def start_scatter(t):
            b = t % NB
            pltpu.async_copy(rows_v.at[b],
                             acc.at[dst_v.at[(t // SUP) % 2, t % SUP]],
                             ssem.at[b], add=True)

        def wait_scatter(t):
            b = t % NB
            pltpu.make_async_copy(rows_v.at[b],
                                  acc.at[dst_v.at[(t // SUP) % 2, t % SUP]],
                                  ssem.at[b]).wait()

        nvec = ZR * (width // 16)
        z16 = jnp.zeros((16,), jnp.float32)

        def zb_body(t, _):
            r = t // (width // 16)
            c = t % (width // 16)
            zbuf[r, pl.ds(c * 16, 16)] = z16
            return 0
        lax.fori_loop(0, nvec, zb_body, 0)

        row0 = sid * ROWS_PER_TILE
        my_rows = jnp.where(sid == NS - 1, N - (NS - 1) * ROWS_PER_TILE,
                            ROWS_PER_TILE)
        ncopies = my_rows // ZR

        def zc_body(t, _):
            pltpu.sync_copy(zbuf, acc.at[pl.ds(row0 + t * ZR, ZR)])
            return 0
        lax.fori_loop(0, ncopies, zc_body, 0)

        plsc.subcore_barrier()

        start_idx(0, 0)
        wait_idx(0, 0)
        start_idx(1, 1)
        for tp in range(GA):
            start_gather(tp)

        def body(t, _):
            tg = t + GA

            @pl.when(tg < T_PER_W)
            def _():
                @pl.when(tg % SUP == 0)
                def _():
                    sg = tg // SUP
                    wait_idx(sg, sg % 2)

                @pl.when(tg >= NB)
                def _():
                    wait_scatter(tg - NB)
                start_gather(tg)

            @pl.when(jnp.logical_and(t % SUP == NB - GA, t > SUP))
            def _():
                s_next = t // SUP + 1

                @pl.when(s_next < NSUP)
                def _():
                    start_idx(s_next, s_next % 2)

            wait_gather(t)
            start_scatter(t)
            return 0
        lax.fori_loop(0, T_PER_W, body, 0)

        for k in range(NB):
            wait_scatter(T_PER_W - NB + k)

        plsc.subcore_barrier()

        def oc_body(t, _):
            pltpu.sync_copy(acc.at[pl.ds(row0 + t * ZR, ZR)],
                            out_hbm.at[pl.ds(cid * N + row0 + t * ZR, ZR)])
            return 0
        lax.fori_loop(0, ncopies, oc_body, 0)

    return segsum


def _mlp_body(h_ref, a0_ref, a1_ref, w1_ref, b1_ref, w2_ref, b2_ref,
              batch_ref, o_ref, y_ref):
    z = h_ref[...] + a0_ref[...] + a1_ref[...]
    t = jnp.dot(z, w1_ref[...], preferred_element_type=jnp.float32)
    t = jnp.maximum(t + b1_ref[...], 0.0)
    t = jnp.dot(t, w2_ref[...], preferred_element_type=jnp.float32)
    hn = jnp.maximum(t + b2_ref[...], 0.0)
    o_ref[...] = hn
    bt = batch_ref[0]
    ids = lax.broadcasted_iota(jnp.int32, (1, B), 1)
    mask = (bt.reshape(BLK, 1) == ids).astype(jnp.float32)
    contrib = lax.dot_general(mask, hn, (((0,), (0,)), ((), ())),
                              preferred_element_type=jnp.float32)

    @pl.when(pl.program_id(0) == 0)
    def _():
        y_ref[...] = jnp.zeros_like(y_ref)
    y_ref[...] += contrib


def _gin_mlp(h, agg2, batch3, w1, b1, w2, b2):
    din = h.shape[1]
    full = lambda shape: pl.BlockSpec(shape, lambda i: (0, 0))
    return pl.pallas_call(
        _mlp_body,
        grid=(NBLK,),
        in_specs=[
            pl.BlockSpec((BLK, din), lambda i: (i, 0)),
            pl.BlockSpec((BLK, din), lambda i: (i, 0)),
            pl.BlockSpec((BLK, din), lambda i: (i + NBLK, 0)),
            full((din, H)), full((1, H)), full((H, H)), full((1, H)),
            pl.BlockSpec((1, 1, BLK), lambda i: (i, 0, 0)),
        ],
        out_specs=[pl.BlockSpec((BLK, H), lambda i: (i, 0)),
                   pl.BlockSpec((B, H), lambda i: (0, 0))],
        out_shape=[jax.ShapeDtypeStruct((N, H), jnp.float32),
                   jax.ShapeDtypeStruct((B, H), jnp.float32)],
    )(h, agg2, agg2, w1, b1.reshape(1, H), w2, b2.reshape(1, H), batch3)


def _ff(z, w0, b0, w1, b1, w2, b2, ws, bs):
    t = jnp.maximum(jnp.dot(z, w0, preferred_element_type=jnp.float32) + b0, 0.0)
    t = jnp.maximum(jnp.dot(t, w1, preferred_element_type=jnp.float32) + b1, 0.0)
    t = jnp.maximum(jnp.dot(t, w2, preferred_element_type=jnp.float32) + b2, 0.0)
    return t + jnp.dot(z, ws, preferred_element_type=jnp.float32) + bs


def _genc_body(y1, y2, y3, y4, w0, b0, w1, b1, w2, b2, ws, bs, o_ref):
    y = jnp.concatenate([y1[...], y2[...], y3[...], y4[...]], axis=1)
    o_ref[...] = _ff(y, w0[...], b0[...], w1[...], b1[...],
                     w2[...], b2[...], ws[...], bs[...])


def _genc(ys, p):
    args = (*ys, p['ff_g_W0'], p['ff_g_b0'].reshape(1, EMB),
            p['ff_g_W1'], p['ff_g_b1'].reshape(1, EMB),
            p['ff_g_W2'], p['ff_g_b2'].reshape(1, EMB),
            p['ff_g_Ws'], p['ff_g_bs'].reshape(1, EMB))
    return pl.pallas_call(
        _genc_body,
        out_shape=jax.ShapeDtypeStruct((B, EMB), jnp.float32),
    )(*args)


def _loss_body(h1, h2, h3, h4, batch_ref, g_ref,
               w0, b0, w1, b1, w2, b2, ws, bs, o_ref):
    l = jnp.concatenate([h1[...], h2[...], h3[...], h4[...]], axis=1)
    le = _ff(l, w0[...], b0[...], w1[...], b1[...], w2[...], b2[...],
             ws[...], bs[...])
    res = lax.dot_general(le, g_ref[...], (((1,), (1,)), ((), ())),
                          preferred_element_type=jnp.float32)
    bt = batch_ref[0]
    ids = lax.broadcasted_iota(jnp.int32, (1, B), 1)
    m = (bt.reshape(BLK, 1) == ids).astype(jnp.float32)
    sp = jnp.maximum(-res, 0.0) + jnp.log1p(jnp.exp(-jnp.abs(res)))
    pos_sum = jnp.sum((_LOG2 - sp) * m)
    neg_sum = jnp.sum((sp + res - _LOG2) * (1.0 - m))
    contrib = neg_sum / (N * (B - 1)) - pos_sum / N

    @pl.when(pl.program_id(0) == 0)
    def _():
        o_ref[...] = jnp.zeros_like(o_ref)
    o_ref[...] += contrib.reshape(1, 1)


def _loss(h1, h2, h3, h4, batch3, g_enc, p):
    hspec = pl.BlockSpec((BLK, H), lambda i: (i, 0))
    full = lambda shape: pl.BlockSpec(shape, lambda i: (0, 0))
    args = (h1, h2, h3, h4, batch3, g_enc,
            p['ff_l_W0'], p['ff_l_b0'].reshape(1, EMB),
            p['ff_l_W1'], p['ff_l_b1'].reshape(1, EMB),
            p['ff_l_W2'], p['ff_l_b2'].reshape(1, EMB),
            p['ff_l_Ws'], p['ff_l_bs'].reshape(1, EMB))
    return pl.pallas_call(
        _loss_body,
        grid=(NBLK,),
        in_specs=[hspec, hspec, hspec, hspec,
                  pl.BlockSpec((1, 1, BLK), lambda i: (i, 0, 0)),
                  full((B, EMB)),
                  full((EMB, EMB)), full((1, EMB)),
                  full((EMB, EMB)), full((1, EMB)),
                  full((EMB, EMB)), full((1, EMB)),
                  full((EMB, EMB)), full((1, EMB))],
        out_specs=pl.BlockSpec((1, 1), lambda i: (0, 0)),
        out_shape=jax.ShapeDtypeStruct((1, 1), jnp.float32),
    )(*args)


def kernel(x, edge_index, batch, edge_attr, params):
    ei_d = edge_index.reshape(2, E // 100, 100)
    ei_h = edge_index.reshape(2, E // 125, 125)
    batch3 = batch.reshape(NBLK, 1, BLK)

    h = x
    hs = []
    ys = []
    for i in range(L):
        segsum = _make_segsum(D if i == 0 else H)
        agg2 = segsum(h, ei_d if i == 0 else ei_h)
        h, y_i = _gin_mlp(h, agg2, batch3,
                          params['gin_W1_%d' % i], params['gin_b1_%d' % i],
                          params['gin_W2_%d' % i], params['gin_b2_%d' % i])
        hs.append(h)
        ys.append(y_i)

    h1, h2, h3, h4 = hs
    g_enc = _genc(ys, params)
    out = _loss(h1, h2, h3, h4, batch3, g_enc, params)
    return out.reshape(())


# speedup vs baseline: 1.0617x; 1.0542x over previous
import functools

import jax
import jax.numpy as jnp
from jax import lax
from jax.experimental import pallas as pl
from jax.experimental.pallas import tpu as pltpu
from jax.experimental.pallas import tpu_sc as plsc

N = 10000
E = 320000
D = 128
H = 64
L = 4
B = 128
EMB = 256

NC = 2
NS = 16
NW = NC * NS

ZR = 16
ROWS_PER_TILE = 640

BLK = 400
NBLK = N // BLK
BLKR = BLK // 2

_LOG2 = 0.6931471805599453


@functools.cache
def _make_segsum(width):
    if width == D:
        CH, SUP, NB, GA = 100, 10, 3, 2
    else:
        CH, SUP, NB, GA = 125, 8, 6, 4
    T_PER_W = E // (NW * CH)
    NSUP = T_PER_W // SUP
    mesh = plsc.VectorSubcoreMesh(
        core_axis_name="c", subcore_axis_name="s", num_cores=NC, num_subcores=NS)

    @functools.partial(
        pl.kernel,
        out_type=jax.ShapeDtypeStruct((2 * N, width), jnp.float32),
        mesh=mesh,
        compiler_params=pltpu.CompilerParams(use_tc_tiling_on_sc=False),
        scratch_types=[
            pltpu.VMEM((2, SUP, CH), jnp.int32),
            pltpu.VMEM((2, SUP, CH), jnp.int32),
            pltpu.VMEM((NB, CH, width), jnp.float32),
            pltpu.VMEM((ZR, width), jnp.float32),
            pltpu.VMEM_SHARED((N, width), jnp.float32),
            pltpu.SemaphoreType.DMA((2,)),
            pltpu.SemaphoreType.DMA((2,)),
            pltpu.SemaphoreType.DMA((NB,)),
            pltpu.SemaphoreType.DMA((NB,)),
        ],
    )
    def segsum(h_hbm, ei_hbm, out_hbm,
               src_v, dst_v, rows_v, zbuf, acc, isem_s, isem_d, gsem, ssem):
        cid = lax.axis_index("c")
        sid = lax.axis_index("s")
        wid = sid * NC + cid
        crow0 = wid * T_PER_W

        def start_idx(s, par):
            r = crow0 + s * SUP
            pltpu.async_copy(ei_hbm.at[0, pl.ds(r, SUP)], src_v.at[par],
                             isem_s.at[par])
            pltpu.async_copy(ei_hbm.at[1, pl.ds(r, SUP)], dst_v.at[par],
                             isem_d.at[par])

        def wait_idx(s, par):
            r = crow0 + s * SUP
            pltpu.make_async_copy(ei_hbm.at[0, pl.ds(r, SUP)], src_v.at[par],
                                  isem_s.at[par]).wait()
            pltpu.make_async_copy(ei_hbm.at[1, pl.ds(r, SUP)], dst_v.at[par],
                                  isem_d.at[par]).wait()

        def start_gather(tg):
            sg = tg // SUP
            parg = sg % 2
            b = tg % NB
            pltpu.async_copy(h_hbm.at[src_v.at[parg, tg % SUP]],
                             rows_v.at[b], gsem.at[b])

        def wait_gather(t):
            b = t % NB
            pltpu.make_async_copy(h_hbm.at[src_v.at[(t // SUP) % 2, t % SUP]],
                                  rows_v.at[b], gsem.at[b]).wait()

        def start_scatter(t):
            b = t % NB
            pltpu.async_copy(rows_v.at[b],
                             acc.at[dst_v.at[(t // SUP) % 2, t % SUP]],
                             ssem.at[b], add=True)

        def wait_scatter(t):
            b = t % NB
            pltpu.make_async_copy(rows_v.at[b],
                                  acc.at[dst_v.at[(t // SUP) % 2, t % SUP]],
                                  ssem.at[b]).wait()

        nvec = ZR * (width // 16)
        z16 = jnp.zeros((16,), jnp.float32)

        def zb_body(t, _):
            r = t // (width // 16)
            c = t % (width // 16)
            zbuf[r, pl.ds(c * 16, 16)] = z16
            return 0
        lax.fori_loop(0, nvec, zb_body, 0)

        row0 = sid * ROWS_PER_TILE
        my_rows = jnp.where(sid == NS - 1, N - (NS - 1) * ROWS_PER_TILE,
                            ROWS_PER_TILE)
        ncopies = my_rows // ZR

        def zc_body(t, _):
            pltpu.sync_copy(zbuf, acc.at[pl.ds(row0 + t * ZR, ZR)])
            return 0
        lax.fori_loop(0, ncopies, zc_body, 0)

        plsc.subcore_barrier()

        start_idx(0, 0)
        wait_idx(0, 0)
        start_idx(1, 1)
        for tp in range(GA):
            start_gather(tp)

        def body(t, _):
            tg = t + GA

            @pl.when(tg < T_PER_W)
            def _():
                @pl.when(tg % SUP == 0)
                def _():
                    sg = tg // SUP
                    wait_idx(sg, sg % 2)

                @pl.when(tg >= NB)
                def _():
                    wait_scatter(tg - NB)
                start_gather(tg)

            @pl.when(jnp.logical_and(t % SUP == NB - GA, t > SUP))
            def _():
                s_next = t // SUP + 1

                @pl.when(s_next < NSUP)
                def _():
                    start_idx(s_next, s_next % 2)

            wait_gather(t)
            start_scatter(t)
            return 0
        lax.fori_loop(0, T_PER_W, body, 0)

        for k in range(NB):
            wait_scatter(T_PER_W - NB + k)

        plsc.subcore_barrier()

        def oc_body(t, _):
            pltpu.sync_copy(acc.at[pl.ds(row0 + t * ZR, ZR)],
                            out_hbm.at[pl.ds(cid * N + row0 + t * ZR, ZR)])
            return 0
        lax.fori_loop(0, ncopies, oc_body, 0)

    return segsum


def _mlp_body(h_ref, a0_ref, a1_ref, w1_ref, b1_ref, w2_ref, b2_ref,
              batch_ref, o_ref, y_ref):
    z = h_ref[...] + a0_ref[...] + a1_ref[...]
    t = jnp.dot(z, w1_ref[...], preferred_element_type=jnp.float32)
    t = jnp.maximum(t + b1_ref[...], 0.0)
    t = jnp.dot(t, w2_ref[...], preferred_element_type=jnp.float32)
    hn = jnp.maximum(t + b2_ref[...], 0.0)
    o_ref[...] = hn
    bp = batch_ref[0, 0]
    ids = lax.broadcasted_iota(jnp.int32, (1, B), 1)
    me = (bp[:, 0:1] == ids).astype(jnp.float32)
    mo = (bp[:, 1:2] == ids).astype(jnp.float32)
    contrib = (
        lax.dot_general(me, hn[:, :H], (((0,), (0,)), ((), ())),
                        preferred_element_type=jnp.float32)
        + lax.dot_general(mo, hn[:, H:], (((0,), (0,)), ((), ())),
                          preferred_element_type=jnp.float32))

    @pl.when(pl.program_id(0) == 0)
    def _():
        y_ref[...] = jnp.zeros_like(y_ref)
    y_ref[...] += contrib


def _bd(w):
    z = jnp.zeros_like(w)
    return jnp.concatenate([jnp.concatenate([w, z], axis=1),
                            jnp.concatenate([z, w], axis=1)], axis=0)


def _gin_mlp(h_r, agg2, batchp, w1, b1, w2, b2):
    din = w1.shape[0]
    a_r = agg2.reshape(N, 2 * din)
    wbd1 = _bd(w1)
    wbd2 = _bd(w2)
    bb1 = jnp.concatenate([b1, b1]).reshape(1, 2 * H)
    bb2 = jnp.concatenate([b2, b2]).reshape(1, 2 * H)
    full = lambda shape: pl.BlockSpec(shape, lambda i: (0, 0))
    return pl.pallas_call(
        _mlp_body,
        grid=(NBLK,),
        in_specs=[
            pl.BlockSpec((BLKR, 2 * din), lambda i: (i, 0)),
            pl.BlockSpec((BLKR, 2 * din), lambda i: (i, 0)),
            pl.BlockSpec((BLKR, 2 * din), lambda i: (i + NBLK, 0)),
            full((2 * din, 2 * H)), full((1, 2 * H)),
            full((2 * H, 2 * H)), full((1, 2 * H)),
            pl.BlockSpec((1, 1, BLKR, 2), lambda i: (i, 0, 0, 0)),
        ],
        out_specs=[pl.BlockSpec((BLKR, 2 * H), lambda i: (i, 0)),
                   pl.BlockSpec((B, H), lambda i: (0, 0))],
        out_shape=[jax.ShapeDtypeStruct((N // 2, 2 * H), jnp.float32),
                   jax.ShapeDtypeStruct((B, H), jnp.float32)],
    )(h_r, a_r, a_r, wbd1, bb1, wbd2, bb2, batchp)


def _ff(z, w0, b0, w1, b1, w2, b2, ws, bs):
    t = jnp.maximum(jnp.dot(z, w0, preferred_element_type=jnp.float32) + b0, 0.0)
    t = jnp.maximum(jnp.dot(t, w1, preferred_element_type=jnp.float32) + b1, 0.0)
    t = jnp.maximum(jnp.dot(t, w2, preferred_element_type=jnp.float32) + b2, 0.0)
    return t + jnp.dot(z, ws, preferred_element_type=jnp.float32) + bs


def _genc_body(y1, y2, y3, y4, w0, b0, w1, b1, w2, b2, ws, bs, o_ref):
    y = jnp.concatenate([y1[...], y2[...], y3[...], y4[...]], axis=1)
    o_ref[...] = _ff(y, w0[...], b0[...], w1[...], b1[...],
                     w2[...], b2[...], ws[...], bs[...])


def _genc(ys, p):
    args = (*ys, p['ff_g_W0'], p['ff_g_b0'].reshape(1, EMB),
            p['ff_g_W1'], p['ff_g_b1'].reshape(1, EMB),
            p['ff_g_W2'], p['ff_g_b2'].reshape(1, EMB),
            p['ff_g_Ws'], p['ff_g_bs'].reshape(1, EMB))
    return pl.pallas_call(
        _genc_body,
        out_shape=jax.ShapeDtypeStruct((B, EMB), jnp.float32),
    )(*args)


def _loss_body(h1, h2, h3, h4, batch_ref, g_ref,
               w0, b0, w1, b1, w2, b2, ws, bs, o_ref):
    bp = batch_ref[0, 0]
    ids = lax.broadcasted_iota(jnp.int32, (1, B), 1)
    contrib = jnp.zeros((), jnp.float32)
    for half in (0, 1):
        sl = slice(0, H) if half == 0 else slice(H, 2 * H)
        l = jnp.concatenate([h1[:, sl], h2[:, sl], h3[:, sl], h4[:, sl]],
                            axis=1)
        le = _ff(l, w0[...], b0[...], w1[...], b1[...], w2[...], b2[...],
                 ws[...], bs[...])
        res = lax.dot_general(le, g_ref[...], (((1,), (1,)), ((), ())),
                              preferred_element_type=jnp.float32)
        m = (bp[:, half:half + 1] == ids).astype(jnp.float32)
        sp = jnp.maximum(-res, 0.0) + jnp.log1p(jnp.exp(-jnp.abs(res)))
        pos_sum = jnp.sum((_LOG2 - sp) * m)
        neg_sum = jnp.sum((sp + res - _LOG2) * (1.0 - m))
        contrib += neg_sum / (N * (B - 1)) - pos_sum / N

    @pl.when(pl.program_id(0) == 0)
    def _():
        o_ref[...] = jnp.zeros_like(o_ref)
    o_ref[...] += contrib.reshape(1, 1)


def _loss(h1, h2, h3, h4, batchp, g_enc, p):
    hspec = pl.BlockSpec((BLKR, 2 * H), lambda i: (i, 0))
    full = lambda shape: pl.BlockSpec(shape, lambda i: (0, 0))
    args = (h1, h2, h3, h4, batchp, g_enc,
            p['ff_l_W0'], p['ff_l_b0'].reshape(1, EMB),
            p['ff_l_W1'], p['ff_l_b1'].reshape(1, EMB),
            p['ff_l_W2'], p['ff_l_b2'].reshape(1, EMB),
            p['ff_l_Ws'], p['ff_l_bs'].reshape(1, EMB))
    return pl.pallas_call(
        _loss_body,
        grid=(NBLK,),
        in_specs=[hspec, hspec, hspec, hspec,
                  pl.BlockSpec((1, 1, BLKR, 2), lambda i: (i, 0, 0, 0)),
                  full((B, EMB)),
                  full((EMB, EMB)), full((1, EMB)),
                  full((EMB, EMB)), full((1, EMB)),
                  full((EMB, EMB)), full((1, EMB)),
                  full((EMB, EMB)), full((1, EMB))],
        out_specs=pl.BlockSpec((1, 1), lambda i: (0, 0)),
        out_shape=jax.ShapeDtypeStruct((1, 1), jnp.float32),
    )(*args)


def kernel(x, edge_index, batch, edge_attr, params):
    ei_d = edge_index.reshape(2, E // 100, 100)
    ei_h = edge_index.reshape(2, E // 125, 125)
    batchp = batch.reshape(NBLK, 1, BLKR, 2)

    h_r = x.reshape(N // 2, 2 * D)
    hs = []
    ys = []
    for i in range(L):
        din = D if i == 0 else H
        segsum = _make_segsum(din)
        agg2 = segsum(h_r.reshape(N, din), ei_d if i == 0 else ei_h)
        h_r, y_i = _gin_mlp(h_r, agg2, batchp,
                            params['gin_W1_%d' % i], params['gin_b1_%d' % i],
                            params['gin_W2_%d' % i], params['gin_b2_%d' % i])
        hs.append(h_r)
        ys.append(y_i)

    h1, h2, h3, h4 = hs
    g_enc = _genc(ys, params)
    out = _loss(h1, h2, h3, h4, batchp, g_enc, params)
    return out.reshape(())


# speedup vs baseline: 1.0824x; 1.0194x over previous
import functools

import jax
import jax.numpy as jnp
from jax import lax
from jax.experimental import pallas as pl
from jax.experimental.pallas import tpu as pltpu
from jax.experimental.pallas import tpu_sc as plsc

N = 10000
E = 320000
D = 128
H = 64
L = 4
B = 128
EMB = 256

NC = 2
NS = 16
NW = NC * NS

ZR = 16
ROWS_PER_TILE = 640

BLK = 400
NBLK = N // BLK
BLKR = BLK // 2

_LOG2 = 0.6931471805599453


@functools.cache
def _make_segsum(width):
    if width == D:
        CH, SUP, NB, GA = 100, 10, 3, 2
    else:
        CH, SUP, NB, GA = 125, 8, 6, 4
    T_PER_W = E // (NW * CH)
    NSUP = T_PER_W // SUP
    mesh = plsc.VectorSubcoreMesh(
        core_axis_name="c", subcore_axis_name="s", num_cores=NC, num_subcores=NS)

    @functools.partial(
        pl.kernel,
        out_type=jax.ShapeDtypeStruct((2 * N, width), jnp.float32),
        mesh=mesh,
        compiler_params=pltpu.CompilerParams(use_tc_tiling_on_sc=False),
        scratch_types=[
            pltpu.VMEM((2, SUP, CH), jnp.int32),
            pltpu.VMEM((2, SUP, CH), jnp.int32),
            pltpu.VMEM((NB, CH, width), jnp.float32),
            pltpu.VMEM((ZR, width), jnp.float32),
            pltpu.VMEM_SHARED((N, width), jnp.float32),
            pltpu.SemaphoreType.DMA((2,)),
            pltpu.SemaphoreType.DMA((2,)),
            pltpu.SemaphoreType.DMA((NB,)),
            pltpu.SemaphoreType.DMA((NB,)),
        ],
    )
    def segsum(h_hbm, ei_hbm, out_hbm,
               src_v, dst_v, rows_v, zbuf, acc, isem_s, isem_d, gsem, ssem):
        cid = lax.axis_index("c")
        sid = lax.axis_index("s")
        wid = sid * NC + cid
        crow0 = wid * T_PER_W

        def start_idx(s, par):
            r = crow0 + s * SUP
            pltpu.async_copy(ei_hbm.at[0, pl.ds(r, SUP)], src_v.at[par],
                             isem_s.at[par])
            pltpu.async_copy(ei_hbm.at[1, pl.ds(r, SUP)], dst_v.at[par],
                             isem_d.at[par])

        def wait_idx(s, par):
            r = crow0 + s * SUP
            pltpu.make_async_copy(ei_hbm.at[0, pl.ds(r, SUP)], src_v.at[par],
                                  isem_s.at[par]).wait()
            pltpu.make_async_copy(ei_hbm.at[1, pl.ds(r, SUP)], dst_v.at[par],
                                  isem_d.at[par]).wait()

        def start_gather(tg):
            sg = tg // SUP
            parg = sg % 2
            b = tg % NB
            pltpu.async_copy(h_hbm.at[src_v.at[parg, tg % SUP]],
                             rows_v.at[b], gsem.at[b])

        def wait_gather(t):
            b = t % NB
            pltpu.make_async_copy(h_hbm.at[src_v.at[(t // SUP) % 2, t % SUP]],
                                  rows_v.at[b], gsem.at[b]).wait()

        def start_scatter(t):
            b = t % NB
            pltpu.async_copy(rows_v.at[b],
                             acc.at[dst_v.at[(t // SUP) % 2, t % SUP]],
                             ssem.at[b], add=True)

        def wait_scatter(t):
            b = t % NB
            pltpu.make_async_copy(rows_v.at[b],
                                  acc.at[dst_v.at[(t // SUP) % 2, t % SUP]],
                                  ssem.at[b]).wait()

        nvec = ZR * (width // 16)
        z16 = jnp.zeros((16,), jnp.float32)

        def zb_body(t, _):
            r = t // (width // 16)
            c = t % (width // 16)
            zbuf[r, pl.ds(c * 16, 16)] = z16
            return 0
        lax.fori_loop(0, nvec, zb_body, 0)

        row0 = sid * ROWS_PER_TILE
        my_rows = jnp.where(sid == NS - 1, N - (NS - 1) * ROWS_PER_TILE,
                            ROWS_PER_TILE)
        ncopies = my_rows // ZR

        def zc_body(t, _):
            pltpu.sync_copy(zbuf, acc.at[pl.ds(row0 + t * ZR, ZR)])
            return 0
        lax.fori_loop(0, ncopies, zc_body, 0)

        plsc.subcore_barrier()

        start_idx(0, 0)
        wait_idx(0, 0)
        start_idx(1, 1)
        for tp in range(GA):
            start_gather(tp)

        def body(t, _):
            tg = t + GA

            @pl.when(tg < T_PER_W)
            def _():
                @pl.when(tg % SUP == 0)
                def _():
                    sg = tg // SUP
                    wait_idx(sg, sg % 2)

                @pl.when(tg >= NB)
                def _():
                    wait_scatter(tg - NB)
                start_gather(tg)

            @pl.when(jnp.logical_and(t % SUP == NB - GA, t > SUP))
            def _():
                s_next = t // SUP + 1

                @pl.when(s_next < NSUP)
                def _():
                    start_idx(s_next, s_next % 2)

            wait_gather(t)
            start_scatter(t)
            return 0
        lax.fori_loop(0, T_PER_W, body, 0)

        for k in range(NB):
            wait_scatter(T_PER_W - NB + k)

        plsc.subcore_barrier()

        def oc_body(t, _):
            pltpu.sync_copy(acc.at[pl.ds(row0 + t * ZR, ZR)],
                            out_hbm.at[pl.ds(cid * N + row0 + t * ZR, ZR)])
            return 0
        lax.fori_loop(0, ncopies, oc_body, 0)

    return segsum


def _mlp_body(h_ref, a0_ref, a1_ref, w1_ref, b1_ref, w2_ref, b2_ref,
              batch_ref, o_ref, y_ref):
    z = h_ref[...] + a0_ref[...] + a1_ref[...]
    t = jnp.dot(z, w1_ref[...], preferred_element_type=jnp.float32)
    t = jnp.maximum(t + b1_ref[...], 0.0)
    t = jnp.dot(t, w2_ref[...], preferred_element_type=jnp.float32)
    hn = jnp.maximum(t + b2_ref[...], 0.0)
    o_ref[...] = hn
    bp = batch_ref[0, 0]
    ids = lax.broadcasted_iota(jnp.int32, (1, B), 1)
    me = (bp[:, 0:1] == ids).astype(jnp.float32)
    mo = (bp[:, 1:2] == ids).astype(jnp.float32)
    contrib = (
        lax.dot_general(me, hn[:, :H], (((0,), (0,)), ((), ())),
                        preferred_element_type=jnp.float32)
        + lax.dot_general(mo, hn[:, H:], (((0,), (0,)), ((), ())),
                          preferred_element_type=jnp.float32))

    @pl.when(pl.program_id(0) == 0)
    def _():
        y_ref[...] = jnp.zeros_like(y_ref)
    y_ref[...] += contrib


def _mlp0_body(xl_ref, xh_ref, a0l, a0h, a1l, a1h, w1_ref, b1_ref,
               w2_ref, b2_ref, batch_ref, o_ref, y_ref):
    zl = xl_ref[...] + a0l[...] + a1l[...]
    zh = xh_ref[...] + a0h[...] + a1h[...]
    tl = jnp.maximum(jnp.dot(zl, w1_ref[...],
                             preferred_element_type=jnp.float32)
                     + b1_ref[...], 0.0)
    th = jnp.maximum(jnp.dot(zh, w1_ref[...],
                             preferred_element_type=jnp.float32)
                     + b1_ref[...], 0.0)
    tt = jnp.concatenate([tl, th], axis=1)
    t = jnp.dot(tt, w2_ref[...], preferred_element_type=jnp.float32)
    hn = jnp.maximum(t + b2_ref[...], 0.0)
    o_ref[...] = hn
    bp = batch_ref[0, 0]
    ids = lax.broadcasted_iota(jnp.int32, (1, B), 1)
    me = (bp[:, 0:1] == ids).astype(jnp.float32)
    mo = (bp[:, 1:2] == ids).astype(jnp.float32)
    contrib = (
        lax.dot_general(me, hn[:, :H], (((0,), (0,)), ((), ())),
                        preferred_element_type=jnp.float32)
        + lax.dot_general(mo, hn[:, H:], (((0,), (0,)), ((), ())),
                          preferred_element_type=jnp.float32))

    @pl.when(pl.program_id(0) == 0)
    def _():
        y_ref[...] = jnp.zeros_like(y_ref)
    y_ref[...] += contrib


def _gin_mlp0(x, agg2, batchp, w1, b1, w2, b2):
    wbd2 = _bd(w2)
    bb1 = b1.reshape(1, H)
    bb2 = jnp.concatenate([b2, b2]).reshape(1, 2 * H)
    full = lambda shape: pl.BlockSpec(shape, lambda i: (0, 0))
    nh = NBLK
    return pl.pallas_call(
        _mlp0_body,
        grid=(NBLK,),
        in_specs=[
            pl.BlockSpec((BLKR, D), lambda i: (i, 0)),
            pl.BlockSpec((BLKR, D), lambda i: (i + nh, 0)),
            pl.BlockSpec((BLKR, D), lambda i: (i, 0)),
            pl.BlockSpec((BLKR, D), lambda i: (i + nh, 0)),
            pl.BlockSpec((BLKR, D), lambda i: (i + 2 * nh, 0)),
            pl.BlockSpec((BLKR, D), lambda i: (i + 3 * nh, 0)),
            full((D, H)), full((1, H)),
            full((2 * H, 2 * H)), full((1, 2 * H)),
            pl.BlockSpec((1, 1, BLKR, 2), lambda i: (i, 0, 0, 0)),
        ],
        out_specs=[pl.BlockSpec((BLKR, 2 * H), lambda i: (i, 0)),
                   pl.BlockSpec((B, H), lambda i: (0, 0))],
        out_shape=[jax.ShapeDtypeStruct((N // 2, 2 * H), jnp.float32),
                   jax.ShapeDtypeStruct((B, H), jnp.float32)],
    )(x, x, agg2, agg2, agg2, agg2, w1, bb1, wbd2, bb2, batchp)


def _bd(w):
    z = jnp.zeros_like(w)
    return jnp.concatenate([jnp.concatenate([w, z], axis=1),
                            jnp.concatenate([z, w], axis=1)], axis=0)


def _gin_mlp(h_r, agg2, batchp, w1, b1, w2, b2):
    din = w1.shape[0]
    a_r = agg2.reshape(N, 2 * din)
    wbd1 = _bd(w1)
    wbd2 = _bd(w2)
    bb1 = jnp.concatenate([b1, b1]).reshape(1, 2 * H)
    bb2 = jnp.concatenate([b2, b2]).reshape(1, 2 * H)
    full = lambda shape: pl.BlockSpec(shape, lambda i: (0, 0))
    return pl.pallas_call(
        _mlp_body,
        grid=(NBLK,),
        in_specs=[
            pl.BlockSpec((BLKR, 2 * din), lambda i: (i, 0)),
            pl.BlockSpec((BLKR, 2 * din), lambda i: (i, 0)),
            pl.BlockSpec((BLKR, 2 * din), lambda i: (i + NBLK, 0)),
            full((2 * din, 2 * H)), full((1, 2 * H)),
            full((2 * H, 2 * H)), full((1, 2 * H)),
            pl.BlockSpec((1, 1, BLKR, 2), lambda i: (i, 0, 0, 0)),
        ],
        out_specs=[pl.BlockSpec((BLKR, 2 * H), lambda i: (i, 0)),
                   pl.BlockSpec((B, H), lambda i: (0, 0))],
        out_shape=[jax.ShapeDtypeStruct((N // 2, 2 * H), jnp.float32),
                   jax.ShapeDtypeStruct((B, H), jnp.float32)],
    )(h_r, a_r, a_r, wbd1, bb1, wbd2, bb2, batchp)


def _ff(z, w0, b0, w1, b1, w2, b2, ws, bs):
    t = jnp.maximum(jnp.dot(z, w0, preferred_element_type=jnp.float32) + b0, 0.0)
    t = jnp.maximum(jnp.dot(t, w1, preferred_element_type=jnp.float32) + b1, 0.0)
    t = jnp.maximum(jnp.dot(t, w2, preferred_element_type=jnp.float32) + b2, 0.0)
    return t + jnp.dot(z, ws, preferred_element_type=jnp.float32) + bs


def _genc_body(y1, y2, y3, y4, w0, b0, w1, b1, w2, b2, ws, bs, o_ref):
    y = jnp.concatenate([y1[...], y2[...], y3[...], y4[...]], axis=1)
    o_ref[...] = _ff(y, w0[...], b0[...], w1[...], b1[...],
                     w2[...], b2[...], ws[...], bs[...])


def _genc(ys, p):
    args = (*ys, p['ff_g_W0'], p['ff_g_b0'].reshape(1, EMB),
            p['ff_g_W1'], p['ff_g_b1'].reshape(1, EMB),
            p['ff_g_W2'], p['ff_g_b2'].reshape(1, EMB),
            p['ff_g_Ws'], p['ff_g_bs'].reshape(1, EMB))
    return pl.pallas_call(
        _genc_body,
        out_shape=jax.ShapeDtypeStruct((B, EMB), jnp.float32),
    )(*args)


def _loss_body(h1, h2, h3, h4, batch_ref, g_ref,
               w0, b0, w1, b1, w2, b2, ws, bs, o_ref):
    bp = batch_ref[0, 0]
    ids = lax.broadcasted_iota(jnp.int32, (1, B), 1)
    contrib = jnp.zeros((), jnp.float32)
    for half in (0, 1):
        sl = slice(0, H) if half == 0 else slice(H, 2 * H)
        l = jnp.concatenate([h1[:, sl], h2[:, sl], h3[:, sl], h4[:, sl]],
                            axis=1)
        le = _ff(l, w0[...], b0[...], w1[...], b1[...], w2[...], b2[...],
                 ws[...], bs[...])
        res = lax.dot_general(le, g_ref[...], (((1,), (1,)), ((), ())),
                              preferred_element_type=jnp.float32)
        m = (bp[:, half:half + 1] == ids).astype(jnp.float32)
        sp = jnp.maximum(-res, 0.0) + jnp.log1p(jnp.exp(-jnp.abs(res)))
        pos_sum = jnp.sum((_LOG2 - sp) * m)
        neg_sum = jnp.sum((sp + res - _LOG2) * (1.0 - m))
        contrib += neg_sum / (N * (B - 1)) - pos_sum / N

    @pl.when(pl.program_id(0) == 0)
    def _():
        o_ref[...] = jnp.zeros_like(o_ref)
    o_ref[...] += contrib.reshape(1, 1)


def _loss(h1, h2, h3, h4, batchp, g_enc, p):
    hspec = pl.BlockSpec((BLKR, 2 * H), lambda i: (i, 0))
    full = lambda shape: pl.BlockSpec(shape, lambda i: (0, 0))
    args = (h1, h2, h3, h4, batchp, g_enc,
            p['ff_l_W0'], p['ff_l_b0'].reshape(1, EMB),
            p['ff_l_W1'], p['ff_l_b1'].reshape(1, EMB),
            p['ff_l_W2'], p['ff_l_b2'].reshape(1, EMB),
            p['ff_l_Ws'], p['ff_l_bs'].reshape(1, EMB))
    return pl.pallas_call(
        _loss_body,
        grid=(NBLK,),
        in_specs=[hspec, hspec, hspec, hspec,
                  pl.BlockSpec((1, 1, BLKR, 2), lambda i: (i, 0, 0, 0)),
                  full((B, EMB)),
                  full((EMB, EMB)), full((1, EMB)),
                  full((EMB, EMB)), full((1, EMB)),
                  full((EMB, EMB)), full((1, EMB)),
                  full((EMB, EMB)), full((1, EMB))],
        out_specs=pl.BlockSpec((1, 1), lambda i: (0, 0)),
        out_shape=jax.ShapeDtypeStruct((1, 1), jnp.float32),
    )(*args)


def kernel(x, edge_index, batch, edge_attr, params):
    perm = lambda v: 2 * jnp.where(v < N // 2, v, v - N // 2) + (v >= N // 2)
    ei_d = edge_index.reshape(2, E // 100, 100)
    ei_h = perm(edge_index).reshape(2, E // 125, 125)
    batchp = jnp.concatenate(
        [batch[:N // 2].reshape(NBLK, 1, BLKR, 1),
         batch[N // 2:].reshape(NBLK, 1, BLKR, 1)], axis=3)

    hs = []
    ys = []
    h_r = None
    for i in range(L):
        din = D if i == 0 else H
        segsum = _make_segsum(din)
        agg2 = segsum(x if i == 0 else h_r.reshape(N, H),
                      ei_d if i == 0 else ei_h)
        if i == 0:
            h_r, y_i = _gin_mlp0(x, agg2, batchp,
                                 params['gin_W1_0'], params['gin_b1_0'],
                                 params['gin_W2_0'], params['gin_b2_0'])
        else:
            h_r, y_i = _gin_mlp(h_r, agg2, batchp,
                                params['gin_W1_%d' % i],
                                params['gin_b1_%d' % i],
                                params['gin_W2_%d' % i],
                                params['gin_b2_%d' % i])
        hs.append(h_r)
        ys.append(y_i)

    h1, h2, h3, h4 = hs
    g_enc = _genc(ys, params)
    out = _loss(h1, h2, h3, h4, batchp, g_enc, params)
    return out.reshape(())
